# Initial kernel scaffold; baseline (speedup 1.0000x reference)
import jax
import jax.numpy as jnp
from jax.experimental import pallas as pl


def kernel(X, e0, e1):
    raise NotImplementedError("write your pallas kernel here")


# speedup vs baseline: 62.5252x; 62.5252x over previous
import jax
import jax.numpy as jnp
from jax.experimental import pallas as pl

_LANES = 128
_ROWS_PER_BLOCK = 112


def _contract_kernel(x_ref, a0_ref, a1_ref, o_ref):
    x = x_ref[...]
    a0 = a0_ref[...]
    a1 = a1_ref[...]
    r = x.shape[1]
    x0 = x[0:4]
    x1 = x[4:8]
    y1 = a0[:, None] * x0[None] + a1[:, None] * x1[None]
    y1 = y1.reshape(3, 2, 2, r, _LANES)
    y2 = a0[None, :, None] * y1[:, 0][:, None] \
        + a1[None, :, None] * y1[:, 1][:, None]
    out = a0[None, None] * y2[:, :, 0][:, :, None] \
        + a1[None, None] * y2[:, :, 1][:, :, None]
    o_ref[...] = out


def kernel(X, e0, e1):
    V = e0.shape[0]
    chunk = _LANES * _ROWS_PER_BLOCK
    nblk = -(-V // chunk)
    vp = nblk * chunk
    rows = vp // _LANES
    pad = vp - V
    x2 = jnp.pad(X.reshape(8, V), ((0, 0), (0, pad))).reshape(8, rows, _LANES)
    a0 = jnp.pad(e0.T, ((0, 0), (0, pad))).reshape(3, rows, _LANES)
    a1 = jnp.pad(e1.T, ((0, 0), (0, pad))).reshape(3, rows, _LANES)
    out = pl.pallas_call(
        _contract_kernel,
        grid=(nblk,),
        in_specs=[
            pl.BlockSpec((8, _ROWS_PER_BLOCK, _LANES), lambda i: (0, i, 0)),
            pl.BlockSpec((3, _ROWS_PER_BLOCK, _LANES), lambda i: (0, i, 0)),
            pl.BlockSpec((3, _ROWS_PER_BLOCK, _LANES), lambda i: (0, i, 0)),
        ],
        out_specs=pl.BlockSpec((3, 3, 3, _ROWS_PER_BLOCK, _LANES),
                               lambda i: (0, 0, 0, i, 0)),
        out_shape=jax.ShapeDtypeStruct((3, 3, 3, rows, _LANES), jnp.float32),
    )(x2, a0, a1)
    return out.reshape(3, 3, 3, vp)[..., :V]


# speedup vs baseline: 151.4319x; 2.4219x over previous
import jax
import jax.numpy as jnp
from jax.experimental import pallas as pl

_LANES = 128
_BV = 12800


def _contract_kernel(x_ref, a0_ref, a1_ref, o_ref):
    bv = x_ref.shape[-1]
    r = bv // _LANES
    x = x_ref[...].reshape(8, r, _LANES)
    a0 = a0_ref[...].reshape(3, r, _LANES)
    a1 = a1_ref[...].reshape(3, r, _LANES)
    x0 = x[0:4]
    x1 = x[4:8]
    y1 = a0[:, None] * x0[None] + a1[:, None] * x1[None]
    y1 = y1.reshape(3, 2, 2, r, _LANES)
    y2 = a0[None, :, None] * y1[:, 0][:, None] \
        + a1[None, :, None] * y1[:, 1][:, None]
    out = a0[None, None] * y2[:, :, 0][:, :, None] \
        + a1[None, None] * y2[:, :, 1][:, :, None]
    o_ref[...] = out.reshape(3, 3, 3, bv)


def kernel(X, e0, e1):
    V = e0.shape[0]
    nblk = -(-V // _BV)
    x2 = X.reshape(8, V)
    a0 = e0.T
    a1 = e1.T
    out = pl.pallas_call(
        _contract_kernel,
        grid=(nblk,),
        in_specs=[
            pl.BlockSpec((8, _BV), lambda i: (0, i)),
            pl.BlockSpec((3, _BV), lambda i: (0, i)),
            pl.BlockSpec((3, _BV), lambda i: (0, i)),
        ],
        out_specs=pl.BlockSpec((3, 3, 3, _BV), lambda i: (0, 0, 0, i)),
        out_shape=jax.ShapeDtypeStruct((3, 3, 3, V), jnp.float32),
    )(x2, a0, a1)
    return out


# speedup vs baseline: 211.1399x; 1.3943x over previous
import jax
import jax.numpy as jnp
from jax.experimental import pallas as pl

_LANES = 128
_BV = 12800


def _contract_kernel(x_ref, a0_ref, a1_ref, o_ref):
    bv = x_ref.shape[-1]
    r = bv // _LANES
    x = x_ref[...].reshape(8, r, _LANES)
    a0 = a0_ref[...].reshape(3, r, _LANES)
    a1 = a1_ref[...].reshape(3, r, _LANES)
    x0 = x[0:4]
    x1 = x[4:8]
    y1 = a0[:, None] * x0[None] + a1[:, None] * x1[None]
    y1 = y1.reshape(3, 2, 2, r, _LANES)
    y2 = a0[None, :, None] * y1[:, 0][:, None] \
        + a1[None, :, None] * y1[:, 1][:, None]
    out = a0[None, None] * y2[:, :, 0][:, :, None] \
        + a1[None, None] * y2[:, :, 1][:, :, None]
    o_ref[...] = out.reshape(3, 3, 3, bv)


def kernel(X, e0, e1):
    V = e0.shape[0]
    nblk = -(-V // _BV)
    a0 = e0.T
    a1 = e1.T
    out = pl.pallas_call(
        _contract_kernel,
        grid=(nblk,),
        in_specs=[
            pl.BlockSpec((2, 2, 2, _BV), lambda i: (0, 0, 0, i)),
            pl.BlockSpec((3, _BV), lambda i: (0, i)),
            pl.BlockSpec((3, _BV), lambda i: (0, i)),
        ],
        out_specs=pl.BlockSpec((3, 3, 3, _BV), lambda i: (0, 0, 0, i)),
        out_shape=jax.ShapeDtypeStruct((3, 3, 3, V), jnp.float32),
    )(X, a0, a1)
    return out
